# baseline (speedup 1.0000x reference)
import functools

import jax
import jax.numpy as jnp
from jax import lax
from jax.experimental import pallas as pl
from jax.experimental.pallas import tpu as pltpu
from jax.experimental.pallas import tpu_sc as plsc

HIDDEN = 64
NC = 2
NS = 16
NW = NC * NS
CHUNK = 80
K = 5
PANEL = 128


def _sc_transpose(tbl_t, tail128, n_pad):
    n_items = tbl_t.shape[1]
    npan = (n_items // PANEL)
    tail_lo = npan * PANEL
    tail_rows = n_pad - tail_lo
    max_steps = npan // NW + 1
    pairs = (max_steps + 2) // 2
    mesh = plsc.VectorSubcoreMesh(core_axis_name="c", subcore_axis_name="s")

    @functools.partial(
        pl.kernel,
        mesh=mesh,
        out_type=jax.ShapeDtypeStruct((n_pad, PANEL), jnp.float32),
        scratch_types=[
            pltpu.VMEM((HIDDEN, PANEL), jnp.float32),
            pltpu.VMEM((HIDDEN, PANEL), jnp.float32),
            pltpu.VMEM((PANEL, PANEL), jnp.float32),
            pltpu.VMEM((PANEL, PANEL), jnp.float32),
            pltpu.VMEM((tail_rows, PANEL), jnp.float32),
            pltpu.SemaphoreType.DMA,
            pltpu.SemaphoreType.DMA,
            pltpu.SemaphoreType.DMA,
            pltpu.SemaphoreType.DMA,
        ],
        compiler_params=pltpu.CompilerParams(
            use_tc_tiling_on_sc=True, needs_layout_passes=False
        ),
    )
    def k(t_hbm, tail_hbm, out_hbm, in0, in1, o0, o1, tv, i0, i1, w0, w1):
        wid = lax.axis_index("s") * NC + lax.axis_index("c")
        ins = [in0, in1]
        outs = [o0, o1]
        isem = [i0, i1]
        wsem = [w0, w1]

        def issue_in(b, p):
            pltpu.async_copy(
                t_hbm.at[:, pl.ds(p * PANEL, PANEL)], ins[b], isem[b]
            )

        def wait_in(b):
            pltpu.make_async_copy(
                t_hbm.at[:, pl.ds(0, PANEL)], ins[b], isem[b]
            ).wait()

        def issue_out(b, p):
            pltpu.async_copy(outs[b], out_hbm.at[pl.ds(p * PANEL, PANEL)], wsem[b])

        def wait_out(b):
            pltpu.make_async_copy(
                outs[b], out_hbm.at[pl.ds(0, PANEL)], wsem[b]
            ).wait()

        def transpose(b):
            def blk(t, carry):
                k0 = (t // 8) * 16
                c0 = (t % 8) * 16
                rows = c0 + lax.iota(jnp.int32, 16)
                for i in range(16):
                    v = ins[b][k0 + i, pl.ds(c0, 16)]
                    cols = jnp.full((16,), k0 + i, jnp.int32)
                    plsc.store_scatter(outs[b], [rows, cols], v)
                return carry

            lax.fori_loop(0, (PANEL // 16) * (HIDDEN // 16), blk, 0)

        issue_in(0, wid)

        def pair_body(i, carry):
            for b in (0, 1):
                j = 2 * i + b
                p = wid + NW * j

                @pl.when(p < npan)
                def _():
                    wait_in(b)

                    @pl.when(p + NW < npan)
                    def _():
                        issue_in(1 - b, p + NW)

                    @pl.when(j >= 2)
                    def _():
                        wait_out(b)

                    transpose(b)
                    issue_out(b, p)

            return carry

        lax.fori_loop(0, pairs, pair_body, 0)
        wait_out(0)
        wait_out(1)

        @pl.when(wid == 0)
        def _():
            pltpu.sync_copy(tail_hbm, tv)
            pltpu.sync_copy(tv, out_hbm.at[pl.ds(tail_lo, tail_rows)])

    return k(tbl_t, tail128)


def _sc_gather(idx2d, table2, batch, hist):
    rows_per_w = batch * hist // NW
    chunks_per_w = rows_per_w // CHUNK
    groups = chunks_per_w // K
    b_per_group = K * CHUNK // hist
    assert groups % 2 == 0 and K * CHUNK % hist == 0
    mesh = plsc.VectorSubcoreMesh(core_axis_name="c", subcore_axis_name="s")

    @functools.partial(
        pl.kernel,
        mesh=mesh,
        out_type=jax.ShapeDtypeStruct((batch, hist, 2 * HIDDEN), jnp.float32),
        scratch_types=[
            pltpu.VMEM((chunks_per_w, CHUNK), jnp.int32),
            pltpu.VMEM((K * CHUNK, HIDDEN), jnp.float32),
            pltpu.VMEM((K * CHUNK, HIDDEN), jnp.float32),
            pltpu.SemaphoreType.DMA,
            pltpu.SemaphoreType.DMA,
            pltpu.SemaphoreType.DMA,
            pltpu.SemaphoreType.DMA,
        ],
        compiler_params=pltpu.CompilerParams(use_tc_tiling_on_sc=False),
    )
    def k(idx_hbm, table_hbm, out_hbm, idx_v, rows0, rows1, g0, g1, w0, w1):
        wid = lax.axis_index("s") * NC + lax.axis_index("c")
        rows = [rows0, rows1]
        gsem = [g0, g1]
        wsem = [w0, w1]
        pltpu.sync_copy(idx_hbm.at[pl.ds(wid * chunks_per_w, chunks_per_w)], idx_v)
        out_b0 = wid * groups * b_per_group

        def issue_gathers(gi, b):
            for j in range(K):
                pltpu.async_copy(
                    table_hbm.at[idx_v.at[gi * K + j]],
                    rows[b].at[pl.ds(j * CHUNK, CHUNK)],
                    gsem[b],
                )

        def wait_gathers(b):
            pltpu.make_async_copy(
                table_hbm.at[pl.ds(0, K * CHUNK)], rows[b], gsem[b]
            ).wait()

        def issue_write(gi, b):
            for r in range(b_per_group):
                pltpu.async_copy(
                    rows[b].at[pl.ds(r * hist, hist)],
                    out_hbm.at[out_b0 + gi * b_per_group + r, :, pl.ds(0, HIDDEN)],
                    wsem[b],
                )

        def wait_write(b):
            for r in range(b_per_group):
                pltpu.make_async_copy(
                    rows[b].at[pl.ds(r * hist, hist)],
                    out_hbm.at[0, :, pl.ds(0, HIDDEN)],
                    wsem[b],
                ).wait()

        issue_gathers(0, 0)

        def pair_body(i, carry):
            for b in (0, 1):
                gi = 2 * i + b
                wait_gathers(b)
                issue_write(gi, b)

                @pl.when(gi >= 1)
                def _():
                    wait_write(1 - b)

                @pl.when(gi + 1 < groups)
                def _():
                    issue_gathers(gi + 1, 1 - b)

            return carry

        lax.fori_loop(0, groups // 2, pair_body, 0)
        wait_write(1)

    return k(idx2d, table2)


def kernel(item_seq, item_seq_len, item_embeddings):
    batch, hist = item_seq.shape
    n_items = item_embeddings.shape[0]
    n_pad = n_items + ((-n_items) % 8)
    tail_lo = (n_items // PANEL) * PANEL
    tail128 = jnp.pad(
        lax.slice(item_embeddings, (tail_lo, 0), (n_items, HIDDEN)),
        ((0, n_pad - n_items), (0, 2 * HIDDEN - HIDDEN)),
    )
    t128 = _sc_transpose(item_embeddings.T, tail128, n_pad)
    table2 = t128.reshape(2 * n_pad, HIDDEN)
    idx2d = (item_seq * 2).reshape(batch * hist // CHUNK, CHUNK)
    return _sc_gather(idx2d, table2, batch, hist)[:, :, :HIDDEN]


# speedup vs baseline: 1.5371x; 1.5371x over previous
import functools

import jax
import jax.numpy as jnp
from jax import lax
from jax.experimental import pallas as pl
from jax.experimental.pallas import tpu as pltpu
from jax.experimental.pallas import tpu_sc as plsc

HIDDEN = 64
NC = 2
NS = 16
NW = NC * NS
CHUNK = 80
K = 5
PANEL = 128


def _sc_transpose(tbl_t, tail128, n_pad):
    n_items = tbl_t.shape[1]
    npan = (n_items // PANEL)
    tail_lo = npan * PANEL
    tail_rows = n_pad - tail_lo
    max_steps = npan // NW + 1
    pairs = (max_steps + 2) // 2
    mesh = plsc.VectorSubcoreMesh(core_axis_name="c", subcore_axis_name="s")

    @functools.partial(
        pl.kernel,
        mesh=mesh,
        out_type=jax.ShapeDtypeStruct((n_pad, PANEL), jnp.float32),
        scratch_types=[
            pltpu.VMEM((HIDDEN, PANEL), jnp.float32),
            pltpu.VMEM((HIDDEN, PANEL), jnp.float32),
            pltpu.VMEM((PANEL, PANEL), jnp.float32),
            pltpu.VMEM((PANEL, PANEL), jnp.float32),
            pltpu.VMEM((tail_rows, PANEL), jnp.float32),
            pltpu.SemaphoreType.DMA,
            pltpu.SemaphoreType.DMA,
            pltpu.SemaphoreType.DMA,
            pltpu.SemaphoreType.DMA,
        ],
        compiler_params=pltpu.CompilerParams(
            use_tc_tiling_on_sc=True, needs_layout_passes=False
        ),
    )
    def k(t_hbm, tail_hbm, out_hbm, in0, in1, o0, o1, tv, i0, i1, w0, w1):
        wid = lax.axis_index("s") * NC + lax.axis_index("c")
        ins = [in0, in1]
        outs = [o0, o1]
        isem = [i0, i1]
        wsem = [w0, w1]

        def issue_in(b, p):
            pltpu.async_copy(
                t_hbm.at[:, pl.ds(p * PANEL, PANEL)], ins[b], isem[b]
            )

        def wait_in(b):
            pltpu.make_async_copy(
                t_hbm.at[:, pl.ds(0, PANEL)], ins[b], isem[b]
            ).wait()

        def issue_out(b, p):
            pltpu.async_copy(outs[b], out_hbm.at[pl.ds(p * PANEL, PANEL)], wsem[b])

        def wait_out(b):
            pltpu.make_async_copy(
                outs[b], out_hbm.at[pl.ds(0, PANEL)], wsem[b]
            ).wait()

        lanes = lax.iota(jnp.int32, 16)
        perms = [jnp.mod(lanes + i, 16) for i in range(16)]

        def transpose(b):
            def blk(t, carry):
                c0 = t * 16
                cols = c0 + lanes
                for kb in range(HIDDEN // 16):
                    k0 = kb * 16
                    for i in range(16):
                        rows = k0 + perms[i]
                        v = plsc.load_gather(ins[b], [rows, cols])
                        plsc.store_scatter(outs[b], [cols, rows], v)
                return carry

            lax.fori_loop(0, PANEL // 16, blk, 0)

        issue_in(0, wid)

        def pair_body(i, carry):
            for b in (0, 1):
                j = 2 * i + b
                p = wid + NW * j

                @pl.when(p < npan)
                def _():
                    wait_in(b)

                    @pl.when(p + NW < npan)
                    def _():
                        issue_in(1 - b, p + NW)

                    @pl.when(j >= 2)
                    def _():
                        wait_out(b)

                    transpose(b)
                    issue_out(b, p)

            return carry

        lax.fori_loop(0, pairs, pair_body, 0)
        wait_out(0)
        wait_out(1)

        @pl.when(wid == 0)
        def _():
            pltpu.sync_copy(tail_hbm, tv)
            pltpu.sync_copy(tv, out_hbm.at[pl.ds(tail_lo, tail_rows)])

    return k(tbl_t, tail128)


def _sc_gather(idx2d, table2, batch, hist):
    rows_per_w = batch * hist // NW
    chunks_per_w = rows_per_w // CHUNK
    groups = chunks_per_w // K
    b_per_group = K * CHUNK // hist
    assert groups % 2 == 0 and K * CHUNK % hist == 0
    mesh = plsc.VectorSubcoreMesh(core_axis_name="c", subcore_axis_name="s")

    @functools.partial(
        pl.kernel,
        mesh=mesh,
        out_type=jax.ShapeDtypeStruct((batch, hist, 2 * HIDDEN), jnp.float32),
        scratch_types=[
            pltpu.VMEM((chunks_per_w, CHUNK), jnp.int32),
            pltpu.VMEM((K * CHUNK, HIDDEN), jnp.float32),
            pltpu.VMEM((K * CHUNK, HIDDEN), jnp.float32),
            pltpu.SemaphoreType.DMA,
            pltpu.SemaphoreType.DMA,
            pltpu.SemaphoreType.DMA,
            pltpu.SemaphoreType.DMA,
        ],
        compiler_params=pltpu.CompilerParams(use_tc_tiling_on_sc=False),
    )
    def k(idx_hbm, table_hbm, out_hbm, idx_v, rows0, rows1, g0, g1, w0, w1):
        wid = lax.axis_index("s") * NC + lax.axis_index("c")
        rows = [rows0, rows1]
        gsem = [g0, g1]
        wsem = [w0, w1]
        pltpu.sync_copy(idx_hbm.at[pl.ds(wid * chunks_per_w, chunks_per_w)], idx_v)
        out_b0 = wid * groups * b_per_group

        def issue_gathers(gi, b):
            for j in range(K):
                pltpu.async_copy(
                    table_hbm.at[idx_v.at[gi * K + j]],
                    rows[b].at[pl.ds(j * CHUNK, CHUNK)],
                    gsem[b],
                )

        def wait_gathers(b):
            pltpu.make_async_copy(
                table_hbm.at[pl.ds(0, K * CHUNK)], rows[b], gsem[b]
            ).wait()

        def issue_write(gi, b):
            for r in range(b_per_group):
                pltpu.async_copy(
                    rows[b].at[pl.ds(r * hist, hist)],
                    out_hbm.at[out_b0 + gi * b_per_group + r, :, pl.ds(0, HIDDEN)],
                    wsem[b],
                )

        def wait_write(b):
            for r in range(b_per_group):
                pltpu.make_async_copy(
                    rows[b].at[pl.ds(r * hist, hist)],
                    out_hbm.at[0, :, pl.ds(0, HIDDEN)],
                    wsem[b],
                ).wait()

        issue_gathers(0, 0)

        def pair_body(i, carry):
            for b in (0, 1):
                gi = 2 * i + b
                wait_gathers(b)
                issue_write(gi, b)

                @pl.when(gi >= 1)
                def _():
                    wait_write(1 - b)

                @pl.when(gi + 1 < groups)
                def _():
                    issue_gathers(gi + 1, 1 - b)

            return carry

        lax.fori_loop(0, groups // 2, pair_body, 0)
        wait_write(1)

    return k(idx2d, table2)


def kernel(item_seq, item_seq_len, item_embeddings):
    batch, hist = item_seq.shape
    n_items = item_embeddings.shape[0]
    n_pad = n_items + ((-n_items) % 8)
    tail_lo = (n_items // PANEL) * PANEL
    tail128 = jnp.pad(
        lax.slice(item_embeddings, (tail_lo, 0), (n_items, HIDDEN)),
        ((0, n_pad - n_items), (0, 2 * HIDDEN - HIDDEN)),
    )
    t128 = _sc_transpose(item_embeddings.T, tail128, n_pad)
    table2 = t128.reshape(2 * n_pad, HIDDEN)
    idx2d = (item_seq * 2).reshape(batch * hist // CHUNK, CHUNK)
    return _sc_gather(idx2d, table2, batch, hist)[:, :, :HIDDEN]


# speedup vs baseline: 1.8270x; 1.1886x over previous
import functools

import jax
import jax.numpy as jnp
from jax import lax
from jax.experimental import pallas as pl
from jax.experimental.pallas import tpu as pltpu
from jax.experimental.pallas import tpu_sc as plsc

HIDDEN = 64
NC = 2
NS = 16
NW = NC * NS
CHUNK = 80
K = 5
PANEL = 128


def _sc_transpose(tbl_t, tail128, n_pad):
    n_items = tbl_t.shape[1]
    npan = (n_items // PANEL)
    tail_lo = npan * PANEL
    tail_rows = n_pad - tail_lo
    max_steps = npan // NW + 1
    pairs = (max_steps + 2) // 2
    mesh = plsc.VectorSubcoreMesh(core_axis_name="c", subcore_axis_name="s")

    @functools.partial(
        pl.kernel,
        mesh=mesh,
        out_type=jax.ShapeDtypeStruct((n_pad, PANEL), jnp.float32),
        scratch_types=[
            pltpu.VMEM((HIDDEN, PANEL), jnp.float32),
            pltpu.VMEM((HIDDEN, PANEL), jnp.float32),
            pltpu.VMEM((PANEL, PANEL), jnp.float32),
            pltpu.VMEM((PANEL, PANEL), jnp.float32),
            pltpu.VMEM((tail_rows, PANEL), jnp.float32),
            pltpu.SemaphoreType.DMA,
            pltpu.SemaphoreType.DMA,
            pltpu.SemaphoreType.DMA,
            pltpu.SemaphoreType.DMA,
        ],
        compiler_params=pltpu.CompilerParams(
            use_tc_tiling_on_sc=True, needs_layout_passes=False
        ),
    )
    def k(t_hbm, tail_hbm, out_hbm, in0, in1, o0, o1, tv, i0, i1, w0, w1):
        wid = lax.axis_index("s") * NC + lax.axis_index("c")
        ins = [in0, in1]
        outs = [o0, o1]
        isem = [i0, i1]
        wsem = [w0, w1]

        def issue_in(b, p):
            pltpu.async_copy(
                t_hbm.at[:, pl.ds(p * PANEL, PANEL)], ins[b], isem[b]
            )

        def wait_in(b):
            pltpu.make_async_copy(
                t_hbm.at[:, pl.ds(0, PANEL)], ins[b], isem[b]
            ).wait()

        def issue_out(b, p):
            pltpu.async_copy(outs[b], out_hbm.at[pl.ds(p * PANEL, PANEL)], wsem[b])

        def wait_out(b):
            pltpu.make_async_copy(
                outs[b], out_hbm.at[pl.ds(0, PANEL)], wsem[b]
            ).wait()

        lanes = lax.iota(jnp.int32, 16)
        perms = [jnp.mod(lanes + i, 16) for i in range(16)]

        def transpose(b):
            def blk(t, carry):
                c0 = (t // (HIDDEN // 16)) * 16
                k0 = (t % (HIDDEN // 16)) * 16
                cols = c0 + lanes
                for i in range(16):
                    rows = k0 + perms[i]
                    v = plsc.load_gather(ins[b], [rows, cols])
                    plsc.store_scatter(outs[b], [cols, rows], v)
                return carry

            lax.fori_loop(0, (PANEL // 16) * (HIDDEN // 16), blk, 0)

        issue_in(0, wid)

        def pair_body(i, carry):
            for b in (0, 1):
                j = 2 * i + b
                p = wid + NW * j

                @pl.when(p < npan)
                def _():
                    wait_in(b)

                    @pl.when(p + NW < npan)
                    def _():
                        issue_in(1 - b, p + NW)

                    @pl.when(j >= 2)
                    def _():
                        wait_out(b)

                    transpose(b)
                    issue_out(b, p)

            return carry

        lax.fori_loop(0, pairs, pair_body, 0)
        wait_out(0)
        wait_out(1)

        @pl.when(wid == 0)
        def _():
            pltpu.sync_copy(tail_hbm, tv)
            pltpu.sync_copy(tv, out_hbm.at[pl.ds(tail_lo, tail_rows)])

    return k(tbl_t, tail128)


def _sc_gather(idx2d, table2, batch, hist):
    rows_per_w = batch * hist // NW
    chunks_per_w = rows_per_w // CHUNK
    groups = chunks_per_w // K
    b_per_group = K * CHUNK // hist
    assert groups % 2 == 0 and K * CHUNK % hist == 0
    mesh = plsc.VectorSubcoreMesh(core_axis_name="c", subcore_axis_name="s")

    @functools.partial(
        pl.kernel,
        mesh=mesh,
        out_type=jax.ShapeDtypeStruct((batch, hist, 2 * HIDDEN), jnp.float32),
        scratch_types=[
            pltpu.VMEM((chunks_per_w, CHUNK), jnp.int32),
            pltpu.VMEM((K * CHUNK, HIDDEN), jnp.float32),
            pltpu.VMEM((K * CHUNK, HIDDEN), jnp.float32),
            pltpu.SemaphoreType.DMA,
            pltpu.SemaphoreType.DMA,
            pltpu.SemaphoreType.DMA,
            pltpu.SemaphoreType.DMA,
        ],
        compiler_params=pltpu.CompilerParams(use_tc_tiling_on_sc=False),
    )
    def k(idx_hbm, table_hbm, out_hbm, idx_v, rows0, rows1, g0, g1, w0, w1):
        wid = lax.axis_index("s") * NC + lax.axis_index("c")
        rows = [rows0, rows1]
        gsem = [g0, g1]
        wsem = [w0, w1]
        pltpu.sync_copy(idx_hbm.at[pl.ds(wid * chunks_per_w, chunks_per_w)], idx_v)
        out_b0 = wid * groups * b_per_group

        def issue_gathers(gi, b):
            for j in range(K):
                pltpu.async_copy(
                    table_hbm.at[idx_v.at[gi * K + j]],
                    rows[b].at[pl.ds(j * CHUNK, CHUNK)],
                    gsem[b],
                )

        def wait_gathers(b):
            pltpu.make_async_copy(
                table_hbm.at[pl.ds(0, K * CHUNK)], rows[b], gsem[b]
            ).wait()

        def issue_write(gi, b):
            for r in range(b_per_group):
                pltpu.async_copy(
                    rows[b].at[pl.ds(r * hist, hist)],
                    out_hbm.at[out_b0 + gi * b_per_group + r, :, pl.ds(0, HIDDEN)],
                    wsem[b],
                )

        def wait_write(b):
            for r in range(b_per_group):
                pltpu.make_async_copy(
                    rows[b].at[pl.ds(r * hist, hist)],
                    out_hbm.at[0, :, pl.ds(0, HIDDEN)],
                    wsem[b],
                ).wait()

        issue_gathers(0, 0)

        def pair_body(i, carry):
            for b in (0, 1):
                gi = 2 * i + b
                wait_gathers(b)
                issue_write(gi, b)

                @pl.when(gi >= 1)
                def _():
                    wait_write(1 - b)

                @pl.when(gi + 1 < groups)
                def _():
                    issue_gathers(gi + 1, 1 - b)

            return carry

        lax.fori_loop(0, groups // 2, pair_body, 0)
        wait_write(1)

    return k(idx2d, table2)


def kernel(item_seq, item_seq_len, item_embeddings):
    batch, hist = item_seq.shape
    n_items = item_embeddings.shape[0]
    n_pad = n_items + ((-n_items) % 8)
    tail_lo = (n_items // PANEL) * PANEL
    tail128 = jnp.pad(
        lax.slice(item_embeddings, (tail_lo, 0), (n_items, HIDDEN)),
        ((0, n_pad - n_items), (0, 2 * HIDDEN - HIDDEN)),
    )
    t128 = _sc_transpose(item_embeddings.T, tail128, n_pad)
    table2 = t128.reshape(2 * n_pad, HIDDEN)
    idx2d = (item_seq * 2).reshape(batch * hist // CHUNK, CHUNK)
    return _sc_gather(idx2d, table2, batch, hist)[:, :, :HIDDEN]


# speedup vs baseline: 2.2962x; 1.2568x over previous
import functools

import jax
import jax.numpy as jnp
from jax import lax
from jax.experimental import pallas as pl
from jax.experimental.pallas import tpu as pltpu
from jax.experimental.pallas import tpu_sc as plsc

HIDDEN = 64
NC = 2
NS = 16
NW = NC * NS
CHUNK = 80
K = 5
PANEL = 128


def _sc_transpose(tbl_t, tail128, n_pad):
    n_items = tbl_t.shape[1]
    npan = (n_items // PANEL)
    tail_lo = npan * PANEL
    tail_rows = n_pad - tail_lo
    max_steps = npan // NW + 1
    pairs = (max_steps + 2) // 2
    mesh = plsc.VectorSubcoreMesh(core_axis_name="c", subcore_axis_name="s")

    @functools.partial(
        pl.kernel,
        mesh=mesh,
        out_type=jax.ShapeDtypeStruct((n_pad, PANEL), jnp.float32),
        scratch_types=[
            pltpu.VMEM((HIDDEN, PANEL), jnp.float32),
            pltpu.VMEM((HIDDEN, PANEL), jnp.float32),
            pltpu.VMEM((PANEL, PANEL), jnp.float32),
            pltpu.VMEM((PANEL, PANEL), jnp.float32),
            pltpu.VMEM((tail_rows, PANEL), jnp.float32),
            pltpu.SemaphoreType.DMA,
            pltpu.SemaphoreType.DMA,
            pltpu.SemaphoreType.DMA,
            pltpu.SemaphoreType.DMA,
        ],
        compiler_params=pltpu.CompilerParams(
            use_tc_tiling_on_sc=True, needs_layout_passes=False
        ),
    )
    def k(t_hbm, tail_hbm, out_hbm, in0, in1, o0, o1, tv, i0, i1, w0, w1):
        wid = lax.axis_index("s") * NC + lax.axis_index("c")
        ins = [in0, in1]
        outs = [o0, o1]
        isem = [i0, i1]
        wsem = [w0, w1]

        def issue_in(b, p):
            pltpu.async_copy(
                t_hbm.at[:, pl.ds(p * PANEL, PANEL)], ins[b], isem[b]
            )

        def wait_in(b):
            pltpu.make_async_copy(
                t_hbm.at[:, pl.ds(0, PANEL)], ins[b], isem[b]
            ).wait()

        def issue_out(b, p):
            pltpu.async_copy(outs[b], out_hbm.at[pl.ds(p * PANEL, PANEL)], wsem[b])

        def wait_out(b):
            pltpu.make_async_copy(
                outs[b], out_hbm.at[pl.ds(0, PANEL)], wsem[b]
            ).wait()

        lanes = lax.iota(jnp.int32, 16)
        perms = [jnp.mod(lanes + i, 16) for i in range(16)]

        def transpose(b):
            def blk(t, carry):
                c0 = (t // (HIDDEN // 32)) * 16
                k0 = (t % (HIDDEN // 32)) * 32
                cols = c0 + lanes
                for i in range(16):
                    rows_a = k0 + perms[i]
                    rows_b = k0 + 16 + perms[i]
                    va = plsc.load_gather(ins[b], [rows_a, cols])
                    vb = plsc.load_gather(ins[b], [rows_b, cols])
                    plsc.store_scatter(outs[b], [cols, rows_a], va)
                    plsc.store_scatter(outs[b], [cols, rows_b], vb)
                return carry

            lax.fori_loop(0, (PANEL // 16) * (HIDDEN // 32), blk, 0)

        issue_in(0, wid)

        def pair_body(i, carry):
            for b in (0, 1):
                j = 2 * i + b
                p = wid + NW * j

                @pl.when(p < npan)
                def _():
                    wait_in(b)

                    @pl.when(p + NW < npan)
                    def _():
                        issue_in(1 - b, p + NW)

                    @pl.when(j >= 2)
                    def _():
                        wait_out(b)

                    transpose(b)
                    issue_out(b, p)

            return carry

        lax.fori_loop(0, pairs, pair_body, 0)
        wait_out(0)
        wait_out(1)

        @pl.when(wid == 0)
        def _():
            pltpu.sync_copy(tail_hbm, tv)
            pltpu.sync_copy(tv, out_hbm.at[pl.ds(tail_lo, tail_rows)])

    return k(tbl_t, tail128)


def _sc_gather(idx2d, table2, batch, hist):
    rows_per_w = batch * hist // NW
    chunks_per_w = rows_per_w // CHUNK
    groups = chunks_per_w // K
    b_per_group = K * CHUNK // hist
    assert groups % 2 == 0 and K * CHUNK % hist == 0
    mesh = plsc.VectorSubcoreMesh(core_axis_name="c", subcore_axis_name="s")

    @functools.partial(
        pl.kernel,
        mesh=mesh,
        out_type=jax.ShapeDtypeStruct((batch, hist, 2 * HIDDEN), jnp.float32),
        scratch_types=[
            pltpu.VMEM((chunks_per_w, CHUNK), jnp.int32),
            pltpu.VMEM((K * CHUNK, HIDDEN), jnp.float32),
            pltpu.VMEM((K * CHUNK, HIDDEN), jnp.float32),
            pltpu.SemaphoreType.DMA,
            pltpu.SemaphoreType.DMA,
            pltpu.SemaphoreType.DMA,
            pltpu.SemaphoreType.DMA,
        ],
        compiler_params=pltpu.CompilerParams(use_tc_tiling_on_sc=False),
    )
    def k(idx_hbm, table_hbm, out_hbm, idx_v, rows0, rows1, g0, g1, w0, w1):
        wid = lax.axis_index("s") * NC + lax.axis_index("c")
        rows = [rows0, rows1]
        gsem = [g0, g1]
        wsem = [w0, w1]
        pltpu.sync_copy(idx_hbm.at[pl.ds(wid * chunks_per_w, chunks_per_w)], idx_v)
        out_b0 = wid * groups * b_per_group

        def issue_gathers(gi, b):
            for j in range(K):
                pltpu.async_copy(
                    table_hbm.at[idx_v.at[gi * K + j]],
                    rows[b].at[pl.ds(j * CHUNK, CHUNK)],
                    gsem[b],
                )

        def wait_gathers(b):
            pltpu.make_async_copy(
                table_hbm.at[pl.ds(0, K * CHUNK)], rows[b], gsem[b]
            ).wait()

        def issue_write(gi, b):
            for r in range(b_per_group):
                pltpu.async_copy(
                    rows[b].at[pl.ds(r * hist, hist)],
                    out_hbm.at[out_b0 + gi * b_per_group + r, :, pl.ds(0, HIDDEN)],
                    wsem[b],
                )

        def wait_write(b):
            for r in range(b_per_group):
                pltpu.make_async_copy(
                    rows[b].at[pl.ds(r * hist, hist)],
                    out_hbm.at[0, :, pl.ds(0, HIDDEN)],
                    wsem[b],
                ).wait()

        issue_gathers(0, 0)

        def pair_body(i, carry):
            for b in (0, 1):
                gi = 2 * i + b
                wait_gathers(b)
                issue_write(gi, b)

                @pl.when(gi >= 1)
                def _():
                    wait_write(1 - b)

                @pl.when(gi + 1 < groups)
                def _():
                    issue_gathers(gi + 1, 1 - b)

            return carry

        lax.fori_loop(0, groups // 2, pair_body, 0)
        wait_write(1)

    return k(idx2d, table2)


def kernel(item_seq, item_seq_len, item_embeddings):
    batch, hist = item_seq.shape
    n_items = item_embeddings.shape[0]
    n_pad = n_items + ((-n_items) % 8)
    tail_lo = (n_items // PANEL) * PANEL
    tail128 = jnp.pad(
        lax.slice(item_embeddings, (tail_lo, 0), (n_items, HIDDEN)),
        ((0, n_pad - n_items), (0, 2 * HIDDEN - HIDDEN)),
    )
    t128 = _sc_transpose(item_embeddings.T, tail128, n_pad)
    table2 = t128.reshape(2 * n_pad, HIDDEN)
    idx2d = (item_seq * 2).reshape(batch * hist // CHUNK, CHUNK)
    return _sc_gather(idx2d, table2, batch, hist)[:, :, :HIDDEN]


# speedup vs baseline: 2.2991x; 1.0013x over previous
import functools

import jax
import jax.numpy as jnp
from jax import lax
from jax.experimental import pallas as pl
from jax.experimental.pallas import tpu as pltpu
from jax.experimental.pallas import tpu_sc as plsc

HIDDEN = 64
NC = 2
NS = 16
NW = NC * NS
CHUNK = 80
K = 10
PANEL = 128


def _sc_transpose(tbl_t, tail128, n_pad):
    n_items = tbl_t.shape[1]
    npan = (n_items // PANEL)
    tail_lo = npan * PANEL
    tail_rows = n_pad - tail_lo
    max_steps = npan // NW + 1
    pairs = (max_steps + 2) // 2
    mesh = plsc.VectorSubcoreMesh(core_axis_name="c", subcore_axis_name="s")

    @functools.partial(
        pl.kernel,
        mesh=mesh,
        out_type=jax.ShapeDtypeStruct((n_pad, PANEL), jnp.float32),
        scratch_types=[
            pltpu.VMEM((HIDDEN, PANEL), jnp.float32),
            pltpu.VMEM((HIDDEN, PANEL), jnp.float32),
            pltpu.VMEM((PANEL, PANEL), jnp.float32),
            pltpu.VMEM((PANEL, PANEL), jnp.float32),
            pltpu.VMEM((tail_rows, PANEL), jnp.float32),
            pltpu.SemaphoreType.DMA,
            pltpu.SemaphoreType.DMA,
            pltpu.SemaphoreType.DMA,
            pltpu.SemaphoreType.DMA,
        ],
        compiler_params=pltpu.CompilerParams(
            use_tc_tiling_on_sc=True, needs_layout_passes=False
        ),
    )
    def k(t_hbm, tail_hbm, out_hbm, in0, in1, o0, o1, tv, i0, i1, w0, w1):
        wid = lax.axis_index("s") * NC + lax.axis_index("c")
        ins = [in0, in1]
        outs = [o0, o1]
        isem = [i0, i1]
        wsem = [w0, w1]

        def issue_in(b, p):
            pltpu.async_copy(
                t_hbm.at[:, pl.ds(p * PANEL, PANEL)], ins[b], isem[b]
            )

        def wait_in(b):
            pltpu.make_async_copy(
                t_hbm.at[:, pl.ds(0, PANEL)], ins[b], isem[b]
            ).wait()

        def issue_out(b, p):
            pltpu.async_copy(outs[b], out_hbm.at[pl.ds(p * PANEL, PANEL)], wsem[b])

        def wait_out(b):
            pltpu.make_async_copy(
                outs[b], out_hbm.at[pl.ds(0, PANEL)], wsem[b]
            ).wait()

        lanes = lax.iota(jnp.int32, 16)
        perms = [jnp.mod(lanes + i, 16) for i in range(16)]

        def transpose(b):
            def blk(t, carry):
                c0 = (t // (HIDDEN // 32)) * 16
                k0 = (t % (HIDDEN // 32)) * 32
                cols = c0 + lanes
                for i in range(16):
                    rows_a = k0 + perms[i]
                    rows_b = k0 + 16 + perms[i]
                    va = plsc.load_gather(ins[b], [rows_a, cols])
                    vb = plsc.load_gather(ins[b], [rows_b, cols])
                    plsc.store_scatter(outs[b], [cols, rows_a], va)
                    plsc.store_scatter(outs[b], [cols, rows_b], vb)
                return carry

            lax.fori_loop(0, (PANEL // 16) * (HIDDEN // 32), blk, 0)

        issue_in(0, wid)

        def pair_body(i, carry):
            for b in (0, 1):
                j = 2 * i + b
                p = wid + NW * j

                @pl.when(p < npan)
                def _():
                    wait_in(b)

                    @pl.when(p + NW < npan)
                    def _():
                        issue_in(1 - b, p + NW)

                    @pl.when(j >= 2)
                    def _():
                        wait_out(b)

                    transpose(b)
                    issue_out(b, p)

            return carry

        lax.fori_loop(0, pairs, pair_body, 0)
        wait_out(0)
        wait_out(1)

        @pl.when(wid == 0)
        def _():
            pltpu.sync_copy(tail_hbm, tv)
            pltpu.sync_copy(tv, out_hbm.at[pl.ds(tail_lo, tail_rows)])

    return k(tbl_t, tail128)


def _sc_gather(idx2d, table2, batch, hist):
    rows_per_w = batch * hist // NW
    chunks_per_w = rows_per_w // CHUNK
    groups = chunks_per_w // K
    b_per_group = K * CHUNK // hist
    assert groups % 2 == 0 and K * CHUNK % hist == 0
    mesh = plsc.VectorSubcoreMesh(core_axis_name="c", subcore_axis_name="s")

    @functools.partial(
        pl.kernel,
        mesh=mesh,
        out_type=jax.ShapeDtypeStruct((batch, hist, 2 * HIDDEN), jnp.float32),
        scratch_types=[
            pltpu.VMEM((chunks_per_w, CHUNK), jnp.int32),
            pltpu.VMEM((K * CHUNK, HIDDEN), jnp.float32),
            pltpu.VMEM((K * CHUNK, HIDDEN), jnp.float32),
            pltpu.SemaphoreType.DMA,
            pltpu.SemaphoreType.DMA,
            pltpu.SemaphoreType.DMA,
            pltpu.SemaphoreType.DMA,
        ],
        compiler_params=pltpu.CompilerParams(use_tc_tiling_on_sc=False),
    )
    def k(idx_hbm, table_hbm, out_hbm, idx_v, rows0, rows1, g0, g1, w0, w1):
        wid = lax.axis_index("s") * NC + lax.axis_index("c")
        rows = [rows0, rows1]
        gsem = [g0, g1]
        wsem = [w0, w1]
        pltpu.sync_copy(idx_hbm.at[pl.ds(wid * chunks_per_w, chunks_per_w)], idx_v)
        out_b0 = wid * groups * b_per_group

        def issue_gathers(gi, b):
            for j in range(K):
                pltpu.async_copy(
                    table_hbm.at[idx_v.at[gi * K + j]],
                    rows[b].at[pl.ds(j * CHUNK, CHUNK)],
                    gsem[b],
                )

        def wait_gathers(b):
            pltpu.make_async_copy(
                table_hbm.at[pl.ds(0, K * CHUNK)], rows[b], gsem[b]
            ).wait()

        def issue_write(gi, b):
            for r in range(b_per_group):
                pltpu.async_copy(
                    rows[b].at[pl.ds(r * hist, hist)],
                    out_hbm.at[out_b0 + gi * b_per_group + r, :, pl.ds(0, HIDDEN)],
                    wsem[b],
                )

        def wait_write(b):
            for r in range(b_per_group):
                pltpu.make_async_copy(
                    rows[b].at[pl.ds(r * hist, hist)],
                    out_hbm.at[0, :, pl.ds(0, HIDDEN)],
                    wsem[b],
                ).wait()

        issue_gathers(0, 0)

        def pair_body(i, carry):
            for b in (0, 1):
                gi = 2 * i + b
                wait_gathers(b)
                issue_write(gi, b)

                @pl.when(gi >= 1)
                def _():
                    wait_write(1 - b)

                @pl.when(gi + 1 < groups)
                def _():
                    issue_gathers(gi + 1, 1 - b)

            return carry

        lax.fori_loop(0, groups // 2, pair_body, 0)
        wait_write(1)

    return k(idx2d, table2)


def kernel(item_seq, item_seq_len, item_embeddings):
    batch, hist = item_seq.shape
    n_items = item_embeddings.shape[0]
    n_pad = n_items + ((-n_items) % 8)
    tail_lo = (n_items // PANEL) * PANEL
    tail128 = jnp.pad(
        lax.slice(item_embeddings, (tail_lo, 0), (n_items, HIDDEN)),
        ((0, n_pad - n_items), (0, 2 * HIDDEN - HIDDEN)),
    )
    t128 = _sc_transpose(item_embeddings.T, tail128, n_pad)
    table2 = t128.reshape(2 * n_pad, HIDDEN)
    idx2d = (item_seq * 2).reshape(batch * hist // CHUNK, CHUNK)
    return _sc_gather(idx2d, table2, batch, hist)[:, :, :HIDDEN]


# speedup vs baseline: 2.4150x; 1.0504x over previous
import functools

import jax
import jax.numpy as jnp
from jax import lax
from jax.experimental import pallas as pl
from jax.experimental.pallas import tpu as pltpu
from jax.experimental.pallas import tpu_sc as plsc

HIDDEN = 64
NC = 2
NS = 16
NW = NC * NS
CHUNK = 80
K = 10
PANEL = 128


def _sc_transpose(tbl_t, tail128, n_pad):
    n_items = tbl_t.shape[1]
    npan = (n_items // PANEL)
    tail_lo = npan * PANEL
    tail_rows = n_pad - tail_lo
    max_steps = npan // NW + 1
    pairs = (max_steps + 2) // 2
    mesh = plsc.VectorSubcoreMesh(core_axis_name="c", subcore_axis_name="s")

    @functools.partial(
        pl.kernel,
        mesh=mesh,
        out_type=jax.ShapeDtypeStruct((n_pad // 2, PANEL), jnp.float32),
        scratch_types=[
            pltpu.VMEM((HIDDEN, PANEL), jnp.float32),
            pltpu.VMEM((HIDDEN, PANEL), jnp.float32),
            pltpu.VMEM((PANEL // 2, PANEL), jnp.float32),
            pltpu.VMEM((PANEL // 2, PANEL), jnp.float32),
            pltpu.VMEM((tail_rows // 2, PANEL), jnp.float32),
            pltpu.SemaphoreType.DMA,
            pltpu.SemaphoreType.DMA,
            pltpu.SemaphoreType.DMA,
            pltpu.SemaphoreType.DMA,
        ],
        compiler_params=pltpu.CompilerParams(
            use_tc_tiling_on_sc=True, needs_layout_passes=False
        ),
    )
    def k(t_hbm, tail_hbm, out_hbm, in0, in1, o0, o1, tv, i0, i1, w0, w1):
        wid = lax.axis_index("s") * NC + lax.axis_index("c")
        ins = [in0, in1]
        outs = [o0, o1]
        isem = [i0, i1]
        wsem = [w0, w1]

        def issue_in(b, p):
            pltpu.async_copy(
                t_hbm.at[:, pl.ds(p * PANEL, PANEL)], ins[b], isem[b]
            )

        def wait_in(b):
            pltpu.make_async_copy(
                t_hbm.at[:, pl.ds(0, PANEL)], ins[b], isem[b]
            ).wait()

        def issue_out(b, p):
            pltpu.async_copy(
                outs[b], out_hbm.at[pl.ds(p * (PANEL // 2), PANEL // 2)], wsem[b]
            )

        def wait_out(b):
            pltpu.make_async_copy(
                outs[b], out_hbm.at[pl.ds(0, PANEL // 2)], wsem[b]
            ).wait()

        lanes = lax.iota(jnp.int32, 16)
        perms = [jnp.mod(lanes + i, 16) for i in range(16)]
        halflanes = lanes // 2
        par64 = (lanes % 2) * HIDDEN

        def transpose(b):
            def blk(t, carry):
                c0 = (t // (HIDDEN // 32)) * 16
                k0 = (t % (HIDDEN // 32)) * 32
                cols = c0 + lanes
                rows_st = c0 // 2 + halflanes
                for i in range(16):
                    rows_a = k0 + perms[i]
                    rows_b = k0 + 16 + perms[i]
                    va = plsc.load_gather(ins[b], [rows_a, cols])
                    vb = plsc.load_gather(ins[b], [rows_b, cols])
                    plsc.store_scatter(outs[b], [rows_st, rows_a + par64], va)
                    plsc.store_scatter(outs[b], [rows_st, rows_b + par64], vb)
                return carry

            lax.fori_loop(0, (PANEL // 16) * (HIDDEN // 32), blk, 0)

        issue_in(0, wid)

        def pair_body(i, carry):
            for b in (0, 1):
                j = 2 * i + b
                p = wid + NW * j

                @pl.when(p < npan)
                def _():
                    wait_in(b)

                    @pl.when(p + NW < npan)
                    def _():
                        issue_in(1 - b, p + NW)

                    @pl.when(j >= 2)
                    def _():
                        wait_out(b)

                    transpose(b)
                    issue_out(b, p)

            return carry

        lax.fori_loop(0, pairs, pair_body, 0)
        wait_out(0)
        wait_out(1)

        @pl.when(wid == 0)
        def _():
            pltpu.sync_copy(tail_hbm, tv)
            pltpu.sync_copy(tv, out_hbm.at[pl.ds(tail_lo // 2, tail_rows // 2)])

    return k(tbl_t, tail128)


def _sc_gather(idx2d, table2, batch, hist):
    rows_per_w = batch * hist // NW
    chunks_per_w = rows_per_w // CHUNK
    groups = chunks_per_w // K
    b_per_group = K * CHUNK // hist
    assert groups % 2 == 0 and K * CHUNK % hist == 0
    mesh = plsc.VectorSubcoreMesh(core_axis_name="c", subcore_axis_name="s")

    @functools.partial(
        pl.kernel,
        mesh=mesh,
        out_type=jax.ShapeDtypeStruct((batch, hist, 2 * HIDDEN), jnp.float32),
        scratch_types=[
            pltpu.VMEM((chunks_per_w, CHUNK), jnp.int32),
            pltpu.VMEM((K * CHUNK, HIDDEN), jnp.float32),
            pltpu.VMEM((K * CHUNK, HIDDEN), jnp.float32),
            pltpu.SemaphoreType.DMA,
            pltpu.SemaphoreType.DMA,
            pltpu.SemaphoreType.DMA,
            pltpu.SemaphoreType.DMA,
        ],
        compiler_params=pltpu.CompilerParams(use_tc_tiling_on_sc=False),
    )
    def k(idx_hbm, table_hbm, out_hbm, idx_v, rows0, rows1, g0, g1, w0, w1):
        wid = lax.axis_index("s") * NC + lax.axis_index("c")
        rows = [rows0, rows1]
        gsem = [g0, g1]
        wsem = [w0, w1]
        pltpu.sync_copy(idx_hbm.at[pl.ds(wid * chunks_per_w, chunks_per_w)], idx_v)
        out_b0 = wid * groups * b_per_group

        def issue_gathers(gi, b):
            for j in range(K):
                pltpu.async_copy(
                    table_hbm.at[idx_v.at[gi * K + j]],
                    rows[b].at[pl.ds(j * CHUNK, CHUNK)],
                    gsem[b],
                )

        def wait_gathers(b):
            pltpu.make_async_copy(
                table_hbm.at[pl.ds(0, K * CHUNK)], rows[b], gsem[b]
            ).wait()

        def issue_write(gi, b):
            for r in range(b_per_group):
                pltpu.async_copy(
                    rows[b].at[pl.ds(r * hist, hist)],
                    out_hbm.at[out_b0 + gi * b_per_group + r, :, pl.ds(0, HIDDEN)],
                    wsem[b],
                )

        def wait_write(b):
            for r in range(b_per_group):
                pltpu.make_async_copy(
                    rows[b].at[pl.ds(r * hist, hist)],
                    out_hbm.at[0, :, pl.ds(0, HIDDEN)],
                    wsem[b],
                ).wait()

        issue_gathers(0, 0)

        def pair_body(i, carry):
            for b in (0, 1):
                gi = 2 * i + b
                wait_gathers(b)
                issue_write(gi, b)

                @pl.when(gi >= 1)
                def _():
                    wait_write(1 - b)

                @pl.when(gi + 1 < groups)
                def _():
                    issue_gathers(gi + 1, 1 - b)

            return carry

        lax.fori_loop(0, groups // 2, pair_body, 0)
        wait_write(1)

    return k(idx2d, table2)


def kernel(item_seq, item_seq_len, item_embeddings):
    batch, hist = item_seq.shape
    n_items = item_embeddings.shape[0]
    n_pad = n_items + ((-n_items) % 16)
    tail_lo = (n_items // PANEL) * PANEL
    tail128 = jnp.pad(
        lax.slice(item_embeddings, (tail_lo, 0), (n_items, HIDDEN)),
        ((0, n_pad - n_items), (0, 0)),
    ).reshape((n_pad - tail_lo) // 2, 2 * HIDDEN)
    t128 = _sc_transpose(item_embeddings.T, tail128, n_pad)
    table2 = t128.reshape(n_pad, HIDDEN)
    idx2d = item_seq.reshape(batch * hist // CHUNK, CHUNK)
    return _sc_gather(idx2d, table2, batch, hist)[:, :, :HIDDEN]
